# Initial kernel scaffold; baseline (speedup 1.0000x reference)
import jax
import jax.numpy as jnp
from jax.experimental import pallas as pl


def kernel(xyz, feature, sample_idx, W0, b0, W1, b1, W2, b2, V0, c0):
    raise NotImplementedError("write your pallas kernel here")


# speedup vs baseline: 1.0048x; 1.0048x over previous
import functools

import jax
import jax.numpy as jnp
from jax.experimental import pallas as pl
from jax.experimental.pallas import tpu as pltpu

_B, _N, _S, _K, _C = 4, 8192, 2048, 32, 64
_SB = 256
_SE = 256


def _dist_body(q_ref, p_ref, out_ref):
    q = q_ref[0]
    p = p_ref[0]
    qn = jnp.sum(q * q, axis=-1)[:, None]
    pn = jnp.sum(p * p, axis=-1)[None, :]
    d = -2.0 * jax.lax.dot_general(
        q, p, (((1,), (1,)), ((), ())), preferred_element_type=jnp.float32
    )
    out_ref[0] = d + qn + pn


def _mlp_body(np_ref, w0_ref, b0_ref, w1_ref, b1_ref, w2_ref, b2_ref,
              v0_ref, c0_ref, out_ref):
    x = np_ref[...]
    xr = x.reshape(_SE * _K, _C + 3)
    h = jnp.maximum(
        jax.lax.dot_general(xr, w0_ref[...], (((1,), (0,)), ((), ())),
                            preferred_element_type=jnp.float32) + b0_ref[...], 0.0)
    h = jnp.maximum(
        jax.lax.dot_general(h, w1_ref[...], (((1,), (0,)), ((), ())),
                            preferred_element_type=jnp.float32) + b1_ref[...], 0.0)
    h = jnp.maximum(
        jax.lax.dot_general(h, w2_ref[...], (((1,), (0,)), ((), ())),
                            preferred_element_type=jnp.float32) + b2_ref[...], 0.0)
    m = jnp.max(h.reshape(_SE, _K, 128), axis=1)
    o = jnp.maximum(
        jax.lax.dot_general(m, v0_ref[...], (((1,), (0,)), ((), ())),
                            preferred_element_type=jnp.float32) + c0_ref[...], 0.0)
    out_ref[...] = o


def kernel(xyz, feature, sample_idx, W0, b0, W1, b1, W2, b2, V0, c0):
    si = sample_idx.astype(jnp.int32)
    batch = jnp.arange(_B)[:, None, None]
    new_xyz = jnp.take_along_axis(xyz, si[:, :, None], axis=1)

    sqrdists = pl.pallas_call(
        _dist_body,
        grid=(_B, _S // _SB),
        in_specs=[
            pl.BlockSpec((1, _SB, 3), lambda b, s: (b, s, 0)),
            pl.BlockSpec((1, _N, 3), lambda b, s: (b, 0, 0)),
        ],
        out_specs=pl.BlockSpec((1, _SB, _N), lambda b, s: (b, s, 0)),
        out_shape=jax.ShapeDtypeStruct((_B, _S, _N), jnp.float32),
    )(new_xyz, xyz)

    _, idx = jax.lax.top_k(-sqrdists, _K)
    grouped_xyz = xyz[batch, idx]
    xyz_diff = grouped_xyz - new_xyz[:, :, None, :]
    grouped_feature = feature[batch, idx]
    new_points = jnp.concatenate([xyz_diff, grouped_feature], axis=-1)
    np2 = new_points.reshape(_B * _S, _K, _C + 3)

    nblk = (_B * _S) // _SE
    wspec = lambda shape: pl.BlockSpec(shape, lambda i: (0,) * len(shape))
    new_feature = pl.pallas_call(
        _mlp_body,
        grid=(nblk,),
        in_specs=[
            pl.BlockSpec((_SE, _K, _C + 3), lambda i: (i, 0, 0)),
            wspec((_C + 3, 64)), wspec((1, 64)),
            wspec((64, 64)), wspec((1, 64)),
            wspec((64, 128)), wspec((1, 128)),
            wspec((128, 128)), wspec((1, 128)),
        ],
        out_specs=pl.BlockSpec((_SE, 128), lambda i: (i, 0)),
        out_shape=jax.ShapeDtypeStruct((_B * _S, 128), jnp.float32),
    )(np2, W0.T, b0[None, :], W1.T, b1[None, :], W2.T, b2[None, :],
      V0.T, c0[None, :])

    return (new_xyz, new_feature.reshape(_B, _S, 128), sample_idx)


# speedup vs baseline: 2.5530x; 2.5407x over previous
import functools

import jax
import jax.numpy as jnp
from jax import lax
from jax.experimental import pallas as pl
from jax.experimental.pallas import tpu as pltpu
from jax.experimental.pallas import tpu_sc as plsc

_B, _N, _S, _K, _C = 4, 8192, 2048, 32, 64
_SB = 256
_SE = 256
_R = _B * _S
_RW = _N + 128
_DEPTH = 32
_OW = 48
_NWORK = 32
_RPW = _R // _NWORK


def _f2key(x):
    b = lax.bitcast_convert_type(x, jnp.int32)
    return b ^ ((b >> 31) & jnp.int32(0x7FFFFFFF))


def _dist_sel_body(q_ref, p_ref, out_ref):
    q = q_ref[...]
    p = p_ref[0]
    qn = jnp.sum(q * q, axis=-1)[:, None]
    pn = jnp.sum(p * p, axis=-1)[None, :]
    d = qn + pn - 2.0 * jax.lax.dot_general(
        q, p, (((1,), (1,)), ((), ())), preferred_element_type=jnp.float32)
    out_ref[:, 0:_N] = _f2key(d)
    t = jnp.min(d[:, 0:256], axis=-1)
    for g in range(1, 32):
        t = jnp.maximum(t, jnp.min(d[:, g * 256:(g + 1) * 256], axis=-1))
    tk = _f2key(t)[:, None]
    out_ref[:, _N:_N + 16] = jnp.broadcast_to(tk, (_SB, 16))
    out_ref[:, _N + 16:_RW] = jnp.zeros((_SB, _RW - _N - 16), jnp.int32)


def _merge_step(lok, lov, hik, hiv, sk, sv):
    r = lax.rev(sk, (0,))
    rv = lax.rev(sv, (0,))
    c1 = hik <= r
    l1k = jnp.where(c1, hik, r)
    l1v = jnp.where(c1, hiv, rv)
    h1k, h1v = plsc.sort_key_val(l1k, l1v)
    rh = lax.rev(h1k, (0,))
    rhv = lax.rev(h1v, (0,))
    c2 = lok <= rh
    l2k = jnp.where(c2, lok, rh)
    l2v = jnp.where(c2, lov, rhv)
    u2k = jnp.where(c2, rh, lok)
    u2v = jnp.where(c2, rhv, lov)
    lok2, lov2 = plsc.sort_key_val(l2k, l2v)
    hik2, hiv2 = plsc.sort_key_val(u2k, u2v)
    return lok2, lov2, hik2, hiv2


def _select_body(du_hbm, out_hbm, rowbuf, ckey, cidx, oidx, sem):
    wid = lax.axis_index("s") * 2 + lax.axis_index("c")
    maxsplat = jnp.full((16,), jnp.int32(0x7FFFFFFF))
    lane = lax.broadcasted_iota(jnp.int32, (16,), 0)
    lane_base = lane * _DEPTH
    lane_cap = lane_base + (_DEPTH - 1)
    dump = 16 * _DEPTH + lane

    def _pre(v, _):
        ckey[pl.ds(v * 16, 16)] = maxsplat
        return 0
    lax.fori_loop(0, (16 * _DEPTH + 16) // 16, _pre, 0)

    def _row(j, _):
        base = wid * _RPW + j
        pltpu.async_copy(du_hbm.at[base], rowbuf, sem).wait()
        t16 = rowbuf[pl.ds(_N, 16)]

        def _scan(i, cnt):
            for u in range(4):
                off = i * 64 + u * 16
                d = rowbuf[pl.ds(off, 16)]
                mask = d <= t16
                dest = jnp.where(mask, cnt, dump)
                plsc.store_scatter(ckey, [dest], d)
                plsc.store_scatter(cidx, [dest], off + lane)
                cnt = jnp.minimum(cnt + mask.astype(jnp.int32), lane_cap)
            return cnt
        lax.fori_loop(0, _N // 64, _scan, lane_base)

        def _mrg(v, carry):
            lok, lov, hik, hiv = carry
            sk, sv = plsc.sort_key_val(ckey[pl.ds(v * 16, 16)],
                                       cidx[pl.ds(v * 16, 16)])
            return _merge_step(lok, lov, hik, hiv, sk, sv)
        lok, lov, hik, hiv = lax.fori_loop(
            0, _DEPTH, _mrg, (maxsplat, lane, maxsplat, lane))

        oidx[pl.ds(0, 16)] = lov
        oidx[pl.ds(16, 16)] = hiv
        pltpu.sync_copy(oidx, out_hbm.at[base])

        def _clr(v, _):
            ckey[pl.ds(v * 16, 16)] = maxsplat
            return 0
        lax.fori_loop(0, _DEPTH, _clr, 0)
        return 0
    lax.fori_loop(0, _RPW, _row, 0)


def _mlp_body(np_ref, w0_ref, b0_ref, w1_ref, b1_ref, w2_ref, b2_ref,
              v0_ref, c0_ref, out_ref):
    x = np_ref[...]
    xr = x.reshape(_SE * _K, _C + 3)
    h = jnp.maximum(
        jax.lax.dot_general(xr, w0_ref[...], (((1,), (0,)), ((), ())),
                            preferred_element_type=jnp.float32) + b0_ref[...], 0.0)
    h = jnp.maximum(
        jax.lax.dot_general(h, w1_ref[...], (((1,), (0,)), ((), ())),
                            preferred_element_type=jnp.float32) + b1_ref[...], 0.0)
    h = jnp.maximum(
        jax.lax.dot_general(h, w2_ref[...], (((1,), (0,)), ((), ())),
                            preferred_element_type=jnp.float32) + b2_ref[...], 0.0)
    m = jnp.max(h.reshape(_SE, _K, 128), axis=1)
    o = jnp.maximum(
        jax.lax.dot_general(m, v0_ref[...], (((1,), (0,)), ((), ())),
                            preferred_element_type=jnp.float32) + c0_ref[...], 0.0)
    out_ref[...] = o


def kernel(xyz, feature, sample_idx, W0, b0, W1, b1, W2, b2, V0, c0):
    si = sample_idx.astype(jnp.int32)
    batch = jnp.arange(_B)[:, None, None]
    new_xyz = jnp.take_along_axis(xyz, si[:, :, None], axis=1)

    du = pl.pallas_call(
        _dist_sel_body,
        grid=(_R // _SB,),
        in_specs=[
            pl.BlockSpec((_SB, 3), lambda i: (i, 0)),
            pl.BlockSpec((1, _N, 3), lambda i: (i // (_S // _SB), 0, 0)),
        ],
        out_specs=pl.BlockSpec((_SB, _RW), lambda i: (i, 0)),
        out_shape=jax.ShapeDtypeStruct((_R, _RW), jnp.int32),
    )(new_xyz.reshape(_R, 3), xyz)

    select = functools.partial(
        pl.kernel,
        out_type=jax.ShapeDtypeStruct((_R, _OW), jnp.int32),
        mesh=plsc.VectorSubcoreMesh(core_axis_name="c", subcore_axis_name="s"),
        compiler_params=pltpu.CompilerParams(needs_layout_passes=False),
        scratch_types=[
            pltpu.VMEM((_RW,), jnp.int32),
            pltpu.VMEM((16 * _DEPTH + 16,), jnp.int32),
            pltpu.VMEM((16 * _DEPTH + 16,), jnp.int32),
            pltpu.VMEM((_OW,), jnp.int32),
            pltpu.SemaphoreType.DMA,
        ],
    )(_select_body)
    idx = select(du)[:, :_K].reshape(_B, _S, _K)

    grouped_xyz = xyz[batch, idx]
    xyz_diff = grouped_xyz - new_xyz[:, :, None, :]
    grouped_feature = feature[batch, idx]
    new_points = jnp.concatenate([xyz_diff, grouped_feature], axis=-1)
    np2 = new_points.reshape(_R, _K, _C + 3)

    nblk = _R // _SE
    wspec = lambda shape: pl.BlockSpec(shape, lambda i: (0,) * len(shape))
    new_feature = pl.pallas_call(
        _mlp_body,
        grid=(nblk,),
        in_specs=[
            pl.BlockSpec((_SE, _K, _C + 3), lambda i: (i, 0, 0)),
            wspec((_C + 3, 64)), wspec((1, 64)),
            wspec((64, 64)), wspec((1, 64)),
            wspec((64, 128)), wspec((1, 128)),
            wspec((128, 128)), wspec((1, 128)),
        ],
        out_specs=pl.BlockSpec((_SE, 128), lambda i: (i, 0)),
        out_shape=jax.ShapeDtypeStruct((_R, 128), jnp.float32),
    )(np2, W0.T, b0[None, :], W1.T, b1[None, :], W2.T, b2[None, :],
      V0.T, c0[None, :])

    return (new_xyz, new_feature.reshape(_B, _S, 128), sample_idx)


# speedup vs baseline: 13.4477x; 5.2674x over previous
import functools

import jax
import jax.numpy as jnp
from jax import lax
from jax.experimental import pallas as pl
from jax.experimental.pallas import tpu as pltpu
from jax.experimental.pallas import tpu_sc as plsc

_B, _N, _S, _K, _C = 4, 8192, 2048, 32, 64
_SB = 256
_SE = 256
_GB = 2048
_R = _B * _S
_BN = _B * _N
_RW = _N + 128
_DEPTH = 32
_OW = 48
_NWORK = 32
_RPW = _R // _NWORK


def _f2key(x):
    b = lax.bitcast_convert_type(x, jnp.int32)
    return b ^ ((b >> 31) & jnp.int32(0x7FFFFFFF))


def _gpre_body(x_ref, f_ref, w3_ref, w64_ref, b0_ref, g_ref):
    g = jax.lax.dot_general(x_ref[...], w3_ref[...], (((1,), (0,)), ((), ())),
                            preferred_element_type=jnp.float32)
    g += jax.lax.dot_general(f_ref[...], w64_ref[...], (((1,), (0,)), ((), ())),
                             preferred_element_type=jnp.float32)
    g_ref[:, 0:64] = g + b0_ref[...]
    g_ref[:, 64:128] = jnp.zeros((_GB, 64), jnp.float32)


def _dist_sel_body(q_ref, p_ref, w3_ref, out_ref, qp_ref):
    q = q_ref[...]
    p = p_ref[0]
    qn = jnp.sum(q * q, axis=-1)[:, None]
    pn = jnp.sum(p * p, axis=-1)[None, :]
    d = qn + pn - 2.0 * jax.lax.dot_general(
        q, p, (((1,), (1,)), ((), ())), preferred_element_type=jnp.float32)
    out_ref[:, 0:_N] = _f2key(d)
    t = jnp.min(d[:, 0:256], axis=-1)
    for g in range(1, 32):
        t = jnp.maximum(t, jnp.min(d[:, g * 256:(g + 1) * 256], axis=-1))
    tk = _f2key(t)[:, None]
    out_ref[:, _N:_N + 16] = jnp.broadcast_to(tk, (_SB, 16))
    out_ref[:, _N + 16:_RW] = jnp.zeros((_SB, _RW - _N - 16), jnp.int32)
    qp_ref[...] = jax.lax.dot_general(
        q, w3_ref[...], (((1,), (0,)), ((), ())),
        preferred_element_type=jnp.float32)


def _merge_step(lok, lov, hik, hiv, sk, sv):
    r = lax.rev(sk, (0,))
    rv = lax.rev(sv, (0,))
    c1 = hik <= r
    l1k = jnp.where(c1, hik, r)
    l1v = jnp.where(c1, hiv, rv)
    h1k, h1v = plsc.sort_key_val(l1k, l1v)
    rh = lax.rev(h1k, (0,))
    rhv = lax.rev(h1v, (0,))
    c2 = lok <= rh
    l2k = jnp.where(c2, lok, rh)
    l2v = jnp.where(c2, lov, rhv)
    u2k = jnp.where(c2, rh, lok)
    u2v = jnp.where(c2, rhv, lov)
    lok2, lov2 = plsc.sort_key_val(l2k, l2v)
    hik2, hiv2 = plsc.sort_key_val(u2k, u2v)
    return lok2, lov2, hik2, hiv2


def _select_body(du_hbm, out_hbm, rowbuf, ckey, cidx, oidx, sem):
    wid = lax.axis_index("s") * 2 + lax.axis_index("c")
    maxsplat = jnp.full((16,), jnp.int32(0x7FFFFFFF))
    lane = lax.broadcasted_iota(jnp.int32, (16,), 0)
    lane_base = lane * _DEPTH
    lane_cap = lane_base + (_DEPTH - 1)
    dump = 16 * _DEPTH + lane

    def _pre(v, _):
        ckey[pl.ds(v * 16, 16)] = maxsplat
        return 0
    lax.fori_loop(0, (16 * _DEPTH + 16) // 16, _pre, 0)

    def _row(j, _):
        base = wid * _RPW + j
        bofs = (base >> 11) << 13
        pltpu.async_copy(du_hbm.at[base], rowbuf, sem).wait()
        t16 = rowbuf[pl.ds(_N, 16)]
        glane = lane + bofs

        def _scan(i, cnt):
            for u in range(4):
                off = i * 64 + u * 16
                d = rowbuf[pl.ds(off, 16)]
                mask = d <= t16
                dest = jnp.where(mask, cnt, dump)
                plsc.store_scatter(ckey, [dest], d)
                plsc.store_scatter(cidx, [dest], off + glane)
                cnt = jnp.minimum(cnt + mask.astype(jnp.int32), lane_cap)
            return cnt
        lax.fori_loop(0, _N // 64, _scan, lane_base)

        def _mrg(v, carry):
            lok, lov, hik, hiv = carry
            sk, sv = plsc.sort_key_val(ckey[pl.ds(v * 16, 16)],
                                       cidx[pl.ds(v * 16, 16)])
            return _merge_step(lok, lov, hik, hiv, sk, sv)
        lok, lov, hik, hiv = lax.fori_loop(
            0, _DEPTH, _mrg, (maxsplat, lane, maxsplat, lane))

        oidx[pl.ds(0, 16)] = lov
        oidx[pl.ds(16, 16)] = hiv
        oidx[pl.ds(32, 16)] = lane
        pltpu.sync_copy(oidx, out_hbm.at[base])

        def _clr(v, _):
            ckey[pl.ds(v * 16, 16)] = maxsplat
            return 0
        lax.fori_loop(0, _DEPTH, _clr, 0)
        return 0
    lax.fori_loop(0, _RPW, _row, 0)


def _gather_body(g_hbm, idx_hbm, out_hbm, idxv, rows_v, sem):
    wid = lax.axis_index("s") * 2 + lax.axis_index("c")

    def _row(j, _):
        r = wid * _RPW + j
        pltpu.sync_copy(idx_hbm.at[pl.ds(r * _OW, _K)], idxv)
        pltpu.async_copy(g_hbm.at[idxv], rows_v, sem).wait()
        pltpu.sync_copy(rows_v, out_hbm.at[pl.ds(r * _K, _K)])
        return 0
    lax.fori_loop(0, _RPW, _row, 0)


def _mlp_body(gg_ref, qp_ref, w1_ref, b1_ref, w2_ref, b2_ref,
              v0_ref, c0_ref, out_ref):
    x = gg_ref[:, 0:64].reshape(_SE, _K, 64) - qp_ref[...][:, None, :]
    h = jnp.maximum(x.reshape(_SE * _K, 64), 0.0)
    h = jnp.maximum(
        jax.lax.dot_general(h, w1_ref[...], (((1,), (0,)), ((), ())),
                            preferred_element_type=jnp.float32) + b1_ref[...], 0.0)
    h = jnp.maximum(
        jax.lax.dot_general(h, w2_ref[...], (((1,), (0,)), ((), ())),
                            preferred_element_type=jnp.float32) + b2_ref[...], 0.0)
    m = jnp.max(h.reshape(_SE, _K, 128), axis=1)
    o = jnp.maximum(
        jax.lax.dot_general(m, v0_ref[...], (((1,), (0,)), ((), ())),
                            preferred_element_type=jnp.float32) + c0_ref[...], 0.0)
    out_ref[...] = o


def kernel(xyz, feature, sample_idx, W0, b0, W1, b1, W2, b2, V0, c0):
    si = sample_idx.astype(jnp.int32)
    new_xyz = jnp.take_along_axis(xyz, si[:, :, None], axis=1)
    w3 = W0[:, :3].T
    w64 = W0[:, 3:].T

    wspec = lambda shape: pl.BlockSpec(shape, lambda i: (0,) * len(shape))

    g = pl.pallas_call(
        _gpre_body,
        grid=(_BN // _GB,),
        in_specs=[
            pl.BlockSpec((_GB, 3), lambda i: (i, 0)),
            pl.BlockSpec((_GB, _C), lambda i: (i, 0)),
            wspec((3, 64)), wspec((_C, 64)), wspec((1, 64)),
        ],
        out_specs=pl.BlockSpec((_GB, 128), lambda i: (i, 0)),
        out_shape=jax.ShapeDtypeStruct((_BN, 128), jnp.float32),
    )(xyz.reshape(_BN, 3), feature.reshape(_BN, _C), w3, w64, b0[None, :])

    du, qproj = pl.pallas_call(
        _dist_sel_body,
        grid=(_R // _SB,),
        in_specs=[
            pl.BlockSpec((_SB, 3), lambda i: (i, 0)),
            pl.BlockSpec((1, _N, 3), lambda i: (i // (_S // _SB), 0, 0)),
            wspec((3, 64)),
        ],
        out_specs=[
            pl.BlockSpec((_SB, _RW), lambda i: (i, 0)),
            pl.BlockSpec((_SB, 64), lambda i: (i, 0)),
        ],
        out_shape=[
            jax.ShapeDtypeStruct((_R, _RW), jnp.int32),
            jax.ShapeDtypeStruct((_R, 64), jnp.float32),
        ],
    )(new_xyz.reshape(_R, 3), xyz, w3)

    select = functools.partial(
        pl.kernel,
        out_type=jax.ShapeDtypeStruct((_R, _OW), jnp.int32),
        mesh=plsc.VectorSubcoreMesh(core_axis_name="c", subcore_axis_name="s"),
        compiler_params=pltpu.CompilerParams(needs_layout_passes=False),
        scratch_types=[
            pltpu.VMEM((_RW,), jnp.int32),
            pltpu.VMEM((16 * _DEPTH + 16,), jnp.int32),
            pltpu.VMEM((16 * _DEPTH + 16,), jnp.int32),
            pltpu.VMEM((_OW,), jnp.int32),
            pltpu.SemaphoreType.DMA,
        ],
    )(_select_body)
    idx = select(du)

    gather = functools.partial(
        pl.kernel,
        out_type=jax.ShapeDtypeStruct((_R * _K, 128), jnp.float32),
        mesh=plsc.VectorSubcoreMesh(core_axis_name="c", subcore_axis_name="s"),
        compiler_params=pltpu.CompilerParams(needs_layout_passes=False),
        scratch_types=[
            pltpu.VMEM((_K,), jnp.int32),
            pltpu.VMEM((_K, 128), jnp.float32),
            pltpu.SemaphoreType.DMA,
        ],
    )(_gather_body)
    gg = gather(g, idx.reshape(_R * _OW))

    new_feature = pl.pallas_call(
        _mlp_body,
        grid=(_R // _SE,),
        in_specs=[
            pl.BlockSpec((_SE * _K, 128), lambda i: (i, 0)),
            pl.BlockSpec((_SE, 64), lambda i: (i, 0)),
            wspec((64, 64)), wspec((1, 64)),
            wspec((64, 128)), wspec((1, 128)),
            wspec((128, 128)), wspec((1, 128)),
        ],
        out_specs=pl.BlockSpec((_SE, 128), lambda i: (i, 0)),
        out_shape=jax.ShapeDtypeStruct((_R, 128), jnp.float32),
    )(gg, qproj, W1.T, b1[None, :], W2.T, b2[None, :], V0.T, c0[None, :])

    return (new_xyz, new_feature.reshape(_B, _S, 128), sample_idx)


# speedup vs baseline: 20.2198x; 1.5036x over previous
import functools

import jax
import jax.numpy as jnp
from jax import lax
from jax.experimental import pallas as pl
from jax.experimental.pallas import tpu as pltpu
from jax.experimental.pallas import tpu_sc as plsc

_B, _N, _S, _K, _C = 4, 8192, 2048, 32, 64
_SB = 256
_SE = 256
_GB = 2048
_R = _B * _S
_BN = _B * _N
_RW = _N + 128
_DEPTH = 32
_OW = 48
_NWORK = 32
_RPW = _R // _NWORK


def _f2key(x):
    b = lax.bitcast_convert_type(x, jnp.int32)
    return b ^ ((b >> 31) & jnp.int32(0x7FFFFFFF))


def _gpre_body(x_ref, f_ref, w3_ref, w64_ref, b0_ref, g_ref):
    g = jax.lax.dot_general(x_ref[...], w3_ref[...], (((1,), (0,)), ((), ())),
                            preferred_element_type=jnp.float32)
    g += jax.lax.dot_general(f_ref[...], w64_ref[...], (((1,), (0,)), ((), ())),
                             preferred_element_type=jnp.float32)
    g_ref[:, 0:64] = g + b0_ref[...]
    g_ref[:, 64:128] = jnp.zeros((_GB, 64), jnp.float32)


def _dist_sel_body(q_ref, p_ref, w3_ref, out_ref, qp_ref):
    q = q_ref[...]
    p = p_ref[0]
    qn = jnp.sum(q * q, axis=-1)[:, None]
    pn = jnp.sum(p * p, axis=-1)[None, :]
    d = qn + pn - 2.0 * jax.lax.dot_general(
        q, p, (((1,), (1,)), ((), ())), preferred_element_type=jnp.float32)
    out_ref[:, 0:_N] = _f2key(d)
    t = jnp.min(d[:, 0:256], axis=-1)
    for g in range(1, 32):
        t = jnp.maximum(t, jnp.min(d[:, g * 256:(g + 1) * 256], axis=-1))
    tk = _f2key(t)[:, None]
    out_ref[:, _N:_N + 16] = jnp.broadcast_to(tk, (_SB, 16))
    out_ref[:, _N + 16:_RW] = jnp.zeros((_SB, _RW - _N - 16), jnp.int32)
    qp_ref[...] = jax.lax.dot_general(
        q, w3_ref[...], (((1,), (0,)), ((), ())),
        preferred_element_type=jnp.float32)


def _merge_step(lok, lov, hik, hiv, sk, sv):
    r = lax.rev(sk, (0,))
    rv = lax.rev(sv, (0,))
    c1 = hik <= r
    l1k = jnp.where(c1, hik, r)
    l1v = jnp.where(c1, hiv, rv)
    h1k, h1v = plsc.sort_key_val(l1k, l1v)
    rh = lax.rev(h1k, (0,))
    rhv = lax.rev(h1v, (0,))
    c2 = lok <= rh
    l2k = jnp.where(c2, lok, rh)
    l2v = jnp.where(c2, lov, rhv)
    u2k = jnp.where(c2, rh, lok)
    u2v = jnp.where(c2, rhv, lov)
    lok2, lov2 = plsc.sort_key_val(l2k, l2v)
    hik2, hiv2 = plsc.sort_key_val(u2k, u2v)
    return lok2, lov2, hik2, hiv2


def _select_body(du_hbm, g_hbm, gg_hbm, rb, ckey, cidx, oidx, rv,
                 rsem0, rsem1, gsem0, gsem1, osem0, osem1):
    wid = lax.axis_index("s") * 2 + lax.axis_index("c")
    maxsplat = jnp.full((16,), jnp.int32(0x7FFFFFFF))
    lane = lax.broadcasted_iota(jnp.int32, (16,), 0)
    lane_base = lane * _DEPTH
    lane_cap = lane_base + (_DEPTH - 1)
    dump = 16 * _DEPTH + lane
    rsems = (rsem0, rsem1)
    gsems = (gsem0, gsem1)
    osems = (osem0, osem1)

    def _pre(v, _):
        ckey[pl.ds(v * 16, 16)] = maxsplat
        return 0
    lax.fori_loop(0, (16 * _DEPTH + 16) // 16, _pre, 0)

    row0 = wid * _RPW
    pltpu.async_copy(du_hbm.at[row0], rb.at[pl.ds(0, _RW)], rsem0)

    def _one_row(jj, p):
        q = 1 - p
        j = jj * 2 + p
        base = row0 + j
        bofs = (base >> 11) << 13
        rbo = p * _RW
        glane = lane + bofs

        pltpu.make_async_copy(du_hbm.at[base], rb.at[pl.ds(rbo, _RW)],
                              rsems[p]).wait()
        bnext = jnp.minimum(base + 1, _R - 1)
        pltpu.async_copy(du_hbm.at[bnext], rb.at[pl.ds(q * _RW, _RW)],
                         rsems[q])
        t16 = rb[pl.ds(rbo + _N, 16)]

        def _scan(i, cnt):
            for u in range(4):
                off = i * 64 + u * 16
                d = rb[pl.ds(rbo + off, 16)]
                mask = d <= t16
                dest = jnp.where(mask, cnt, dump)
                plsc.store_scatter(ckey, [dest], d)
                plsc.store_scatter(cidx, [dest], off + glane)
                cnt = jnp.minimum(cnt + mask.astype(jnp.int32), lane_cap)
            return cnt
        cnt_end = lax.fori_loop(0, _N // 64, _scan, lane_base)
        mx = jnp.max(cnt_end - lane_base)

        def _mrg_at(soff):
            def _mrg(v, carry):
                sk, sv = plsc.sort_key_val(ckey[pl.ds(v * 32 + soff, 16)],
                                           cidx[pl.ds(v * 32 + soff, 16)])
                return _merge_step(*carry, sk, sv)
            return _mrg
        carry = lax.fori_loop(0, 16, _mrg_at(0),
                              (maxsplat, lane, maxsplat, lane))
        carry = lax.cond(mx > 16,
                         lambda c: lax.fori_loop(0, 16, _mrg_at(16), c),
                         lambda c: c, carry)
        lok, lov, hik, hiv = carry

        oidx[pl.ds(p * _OW, 16)] = lov
        oidx[pl.ds(p * _OW + 16, 16)] = hiv

        def _flush_prev():
            pltpu.make_async_copy(g_hbm.at[oidx.at[pl.ds(q * _OW, _K)]],
                                  rv.at[pl.ds(q * _K, _K)], gsems[q]).wait()
            pltpu.async_copy(rv.at[pl.ds(q * _K, _K)],
                             gg_hbm.at[pl.ds((base - 1) * _K, _K)], osems[q])

        def _wait_own():
            pltpu.make_async_copy(rv.at[pl.ds(p * _K, _K)],
                                  gg_hbm.at[pl.ds(0, _K)], osems[p]).wait()

        if p == 0:
            @pl.when(jj >= 1)
            def _():
                _flush_prev()
                _wait_own()
        else:
            _flush_prev()

            @pl.when(jj >= 1)
            def _():
                _wait_own()

        pltpu.async_copy(g_hbm.at[oidx.at[pl.ds(p * _OW, _K)]],
                         rv.at[pl.ds(p * _K, _K)], gsems[p])

        def _clr(v, _):
            ckey[pl.ds(v * 32, 16)] = maxsplat
            return 0
        lax.fori_loop(0, 16, _clr, 0)

        @pl.when(mx > 16)
        def _():
            def _clro(v, _):
                ckey[pl.ds(v * 32 + 16, 16)] = maxsplat
                return 0
            lax.fori_loop(0, 16, _clro, 0)

    def _pair(jj, _):
        _one_row(jj, 0)
        _one_row(jj, 1)
        return 0
    lax.fori_loop(0, _RPW // 2, _pair, 0)

    last = row0 + _RPW - 1
    pltpu.make_async_copy(g_hbm.at[oidx.at[pl.ds(_OW, _K)]],
                          rv.at[pl.ds(_K, _K)], gsem1).wait()
    pltpu.async_copy(rv.at[pl.ds(_K, _K)],
                     gg_hbm.at[pl.ds(last * _K, _K)], osem1)
    pltpu.make_async_copy(rv.at[pl.ds(0, _K)],
                          gg_hbm.at[pl.ds(0, _K)], osem0).wait()
    pltpu.make_async_copy(rv.at[pl.ds(_K, _K)],
                          gg_hbm.at[pl.ds(0, _K)], osem1).wait()
    pltpu.make_async_copy(du_hbm.at[last], rb.at[pl.ds(0, _RW)],
                          rsems[0]).wait()


def _mlp_body(gg_ref, qp_ref, w1_ref, b1_ref, w2_ref, b2_ref,
              v0_ref, c0_ref, out_ref):
    x = gg_ref[:, 0:64].reshape(_SE, _K, 64) - qp_ref[...][:, None, :]
    h = jnp.maximum(x.reshape(_SE * _K, 64), 0.0)
    h = jnp.maximum(
        jax.lax.dot_general(h, w1_ref[...], (((1,), (0,)), ((), ())),
                            preferred_element_type=jnp.float32) + b1_ref[...], 0.0)
    h = jnp.maximum(
        jax.lax.dot_general(h, w2_ref[...], (((1,), (0,)), ((), ())),
                            preferred_element_type=jnp.float32) + b2_ref[...], 0.0)
    m = jnp.max(h.reshape(_SE, _K, 128), axis=1)
    o = jnp.maximum(
        jax.lax.dot_general(m, v0_ref[...], (((1,), (0,)), ((), ())),
                            preferred_element_type=jnp.float32) + c0_ref[...], 0.0)
    out_ref[...] = o


def kernel(xyz, feature, sample_idx, W0, b0, W1, b1, W2, b2, V0, c0):
    si = sample_idx.astype(jnp.int32)
    new_xyz = jnp.take_along_axis(xyz, si[:, :, None], axis=1)
    w3 = W0[:, :3].T
    w64 = W0[:, 3:].T

    wspec = lambda shape: pl.BlockSpec(shape, lambda i: (0,) * len(shape))

    g = pl.pallas_call(
        _gpre_body,
        grid=(_BN // _GB,),
        in_specs=[
            pl.BlockSpec((_GB, 3), lambda i: (i, 0)),
            pl.BlockSpec((_GB, _C), lambda i: (i, 0)),
            wspec((3, 64)), wspec((_C, 64)), wspec((1, 64)),
        ],
        out_specs=pl.BlockSpec((_GB, 128), lambda i: (i, 0)),
        out_shape=jax.ShapeDtypeStruct((_BN, 128), jnp.float32),
    )(xyz.reshape(_BN, 3), feature.reshape(_BN, _C), w3, w64, b0[None, :])

    du, qproj = pl.pallas_call(
        _dist_sel_body,
        grid=(_R // _SB,),
        in_specs=[
            pl.BlockSpec((_SB, 3), lambda i: (i, 0)),
            pl.BlockSpec((1, _N, 3), lambda i: (i // (_S // _SB), 0, 0)),
            wspec((3, 64)),
        ],
        out_specs=[
            pl.BlockSpec((_SB, _RW), lambda i: (i, 0)),
            pl.BlockSpec((_SB, 64), lambda i: (i, 0)),
        ],
        out_shape=[
            jax.ShapeDtypeStruct((_R, _RW), jnp.int32),
            jax.ShapeDtypeStruct((_R, 64), jnp.float32),
        ],
    )(new_xyz.reshape(_R, 3), xyz, w3)

    select = functools.partial(
        pl.kernel,
        out_type=jax.ShapeDtypeStruct((_R * _K, 128), jnp.float32),
        mesh=plsc.VectorSubcoreMesh(core_axis_name="c", subcore_axis_name="s"),
        compiler_params=pltpu.CompilerParams(needs_layout_passes=False),
        scratch_types=[
            pltpu.VMEM((2 * _RW,), jnp.int32),
            pltpu.VMEM((16 * _DEPTH + 16,), jnp.int32),
            pltpu.VMEM((16 * _DEPTH + 16,), jnp.int32),
            pltpu.VMEM((2 * _OW,), jnp.int32),
            pltpu.VMEM((2 * _K, 128), jnp.float32),
            pltpu.SemaphoreType.DMA,
            pltpu.SemaphoreType.DMA,
            pltpu.SemaphoreType.DMA,
            pltpu.SemaphoreType.DMA,
            pltpu.SemaphoreType.DMA,
            pltpu.SemaphoreType.DMA,
        ],
    )(_select_body)
    gg = select(du, g)

    new_feature = pl.pallas_call(
        _mlp_body,
        grid=(_R // _SE,),
        in_specs=[
            pl.BlockSpec((_SE * _K, 128), lambda i: (i, 0)),
            pl.BlockSpec((_SE, 64), lambda i: (i, 0)),
            wspec((64, 64)), wspec((1, 64)),
            wspec((64, 128)), wspec((1, 128)),
            wspec((128, 128)), wspec((1, 128)),
        ],
        out_specs=pl.BlockSpec((_SE, 128), lambda i: (i, 0)),
        out_shape=jax.ShapeDtypeStruct((_R, 128), jnp.float32),
    )(gg, qproj, W1.T, b1[None, :], W2.T, b2[None, :], V0.T, c0[None, :])

    return (new_xyz, new_feature.reshape(_B, _S, 128), sample_idx)


# speedup vs baseline: 20.3792x; 1.0079x over previous
import functools

import jax
import jax.numpy as jnp
from jax import lax
from jax.experimental import pallas as pl
from jax.experimental.pallas import tpu as pltpu
from jax.experimental.pallas import tpu_sc as plsc

_B, _N, _S, _K, _C = 4, 8192, 2048, 32, 64
_SB = 256
_SE = 256
_GB = 2048
_R = _B * _S
_BN = _B * _N
_RW = _N + 128
_DEPTH = 32
_OW = 48
_NWORK = 32
_RPW = _R // _NWORK


def _f2key(x):
    b = lax.bitcast_convert_type(x, jnp.int32)
    return b ^ ((b >> 31) & jnp.int32(0x7FFFFFFF))


def _gpre_body(x_ref, f_ref, w3_ref, w64_ref, b0_ref, g_ref):
    g = jax.lax.dot_general(x_ref[...], w3_ref[...], (((1,), (0,)), ((), ())),
                            preferred_element_type=jnp.float32)
    g += jax.lax.dot_general(f_ref[...], w64_ref[...], (((1,), (0,)), ((), ())),
                             preferred_element_type=jnp.float32)
    g_ref[:, 0:64] = g + b0_ref[...]
    g_ref[:, 64:128] = jnp.zeros((_GB, 64), jnp.float32)


def _dist_sel_body(q_ref, p_ref, w3_ref, out_ref, qp_ref):
    q = q_ref[...]
    p = p_ref[0]
    qn = jnp.sum(q * q, axis=-1)[:, None]
    pn = jnp.sum(p * p, axis=-1)[None, :]
    d = qn + pn - 2.0 * jax.lax.dot_general(
        q, p, (((1,), (1,)), ((), ())), preferred_element_type=jnp.float32)
    out_ref[:, 0:_N] = _f2key(d)
    t = jnp.min(d[:, 0:256], axis=-1)
    for g in range(1, 32):
        t = jnp.maximum(t, jnp.min(d[:, g * 256:(g + 1) * 256], axis=-1))
    tk = _f2key(t)[:, None]
    out_ref[:, _N:_N + 16] = jnp.broadcast_to(tk, (_SB, 16))
    out_ref[:, _N + 16:_RW] = jnp.zeros((_SB, _RW - _N - 16), jnp.int32)
    qp_ref[...] = jax.lax.dot_general(
        q, w3_ref[...], (((1,), (0,)), ((), ())),
        preferred_element_type=jnp.float32)


def _merge_step(lok, lov, hik, hiv, sk, sv):
    r = lax.rev(sk, (0,))
    rv = lax.rev(sv, (0,))
    c1 = hik <= r
    l1k = jnp.where(c1, hik, r)
    l1v = jnp.where(c1, hiv, rv)
    h1k, h1v = plsc.sort_key_val(l1k, l1v)
    rh = lax.rev(h1k, (0,))
    rhv = lax.rev(h1v, (0,))
    c2 = lok <= rh
    l2k = jnp.where(c2, lok, rh)
    l2v = jnp.where(c2, lov, rhv)
    u2k = jnp.where(c2, rh, lok)
    u2v = jnp.where(c2, rhv, lov)
    lok2, lov2 = plsc.sort_key_val(l2k, l2v)
    hik2, hiv2 = plsc.sort_key_val(u2k, u2v)
    return lok2, lov2, hik2, hiv2


def _select_body(du_hbm, g_hbm, gg_hbm, rb, ckey, cidx, oidx, rv,
                 rsem0, rsem1, gsem0, gsem1, osem0, osem1):
    wid = lax.axis_index("s") * 2 + lax.axis_index("c")
    maxsplat = jnp.full((16,), jnp.int32(0x7FFFFFFF))
    lane = lax.broadcasted_iota(jnp.int32, (16,), 0)
    dump = 16 * _DEPTH + lane
    rsems = (rsem0, rsem1)
    gsems = (gsem0, gsem1)
    osems = (osem0, osem1)

    def _pre(v, _):
        ckey[pl.ds(v * 16, 16)] = maxsplat
        return 0
    lax.fori_loop(0, (16 * _DEPTH + 16) // 16, _pre, 0)

    row0 = wid * _RPW
    pltpu.async_copy(du_hbm.at[row0], rb.at[pl.ds(0, _RW)], rsem0)

    def _one_row(jj, p):
        q = 1 - p
        j = jj * 2 + p
        base = row0 + j
        bofs = (base >> 11) << 13
        rbo = p * _RW
        glane = lane + bofs

        pltpu.make_async_copy(du_hbm.at[base], rb.at[pl.ds(rbo, _RW)],
                              rsems[p]).wait()
        bnext = jnp.minimum(base + 1, _R - 1)
        pltpu.async_copy(du_hbm.at[bnext], rb.at[pl.ds(q * _RW, _RW)],
                         rsems[q])
        t16 = rb[pl.ds(rbo + _N, 16)]

        def _scan(i, cnt):
            for u in range(4):
                off = i * 64 + u * 16
                d = rb[pl.ds(rbo + off, 16)]
                mask = d <= t16
                dest = jnp.where(mask, cnt * 16 + lane, dump)
                plsc.store_scatter(ckey, [dest], d)
                plsc.store_scatter(cidx, [dest], off + glane)
                cnt = jnp.minimum(cnt + mask.astype(jnp.int32), _DEPTH - 1)
            return cnt
        cnt_end = lax.fori_loop(0, _N // 64, _scan,
                                jnp.zeros((16,), jnp.int32))
        mx = jnp.max(cnt_end)
        nv = jnp.minimum(mx + 1, _DEPTH)

        def _mrg(v, carry):
            sk, sv = plsc.sort_key_val(ckey[pl.ds(v * 16, 16)],
                                       cidx[pl.ds(v * 16, 16)])
            return _merge_step(*carry, sk, sv)
        lok, lov, hik, hiv = lax.fori_loop(
            0, nv, _mrg, (maxsplat, lane, maxsplat, lane))

        oidx[pl.ds(p * _OW, 16)] = lov
        oidx[pl.ds(p * _OW + 16, 16)] = hiv

        def _flush_prev():
            pltpu.make_async_copy(g_hbm.at[oidx.at[pl.ds(q * _OW, _K)]],
                                  rv.at[pl.ds(q * _K, _K)], gsems[q]).wait()
            pltpu.async_copy(rv.at[pl.ds(q * _K, _K)],
                             gg_hbm.at[pl.ds((base - 1) * _K, _K)], osems[q])

        def _wait_own():
            pltpu.make_async_copy(rv.at[pl.ds(p * _K, _K)],
                                  gg_hbm.at[pl.ds(0, _K)], osems[p]).wait()

        if p == 0:
            @pl.when(jj >= 1)
            def _():
                _flush_prev()
                _wait_own()
        else:
            _flush_prev()

            @pl.when(jj >= 1)
            def _():
                _wait_own()

        pltpu.async_copy(g_hbm.at[oidx.at[pl.ds(p * _OW, _K)]],
                         rv.at[pl.ds(p * _K, _K)], gsems[p])

        def _clr(v, _):
            ckey[pl.ds(v * 16, 16)] = maxsplat
            return 0
        lax.fori_loop(0, nv, _clr, 0)

    def _pair(jj, _):
        _one_row(jj, 0)
        _one_row(jj, 1)
        return 0
    lax.fori_loop(0, _RPW // 2, _pair, 0)

    last = row0 + _RPW - 1
    pltpu.make_async_copy(g_hbm.at[oidx.at[pl.ds(_OW, _K)]],
                          rv.at[pl.ds(_K, _K)], gsem1).wait()
    pltpu.async_copy(rv.at[pl.ds(_K, _K)],
                     gg_hbm.at[pl.ds(last * _K, _K)], osem1)
    pltpu.make_async_copy(rv.at[pl.ds(0, _K)],
                          gg_hbm.at[pl.ds(0, _K)], osem0).wait()
    pltpu.make_async_copy(rv.at[pl.ds(_K, _K)],
                          gg_hbm.at[pl.ds(0, _K)], osem1).wait()
    pltpu.make_async_copy(du_hbm.at[last], rb.at[pl.ds(0, _RW)],
                          rsems[0]).wait()


def _mlp_body(gg_ref, qp_ref, w1_ref, b1_ref, w2_ref, b2_ref,
              v0_ref, c0_ref, out_ref):
    x = gg_ref[:, 0:64].reshape(_SE, _K, 64) - qp_ref[...][:, None, :]
    h = jnp.maximum(x.reshape(_SE * _K, 64), 0.0)
    h = jnp.maximum(
        jax.lax.dot_general(h, w1_ref[...], (((1,), (0,)), ((), ())),
                            preferred_element_type=jnp.float32) + b1_ref[...], 0.0)
    h = jnp.maximum(
        jax.lax.dot_general(h, w2_ref[...], (((1,), (0,)), ((), ())),
                            preferred_element_type=jnp.float32) + b2_ref[...], 0.0)
    m = jnp.max(h.reshape(_SE, _K, 128), axis=1)
    o = jnp.maximum(
        jax.lax.dot_general(m, v0_ref[...], (((1,), (0,)), ((), ())),
                            preferred_element_type=jnp.float32) + c0_ref[...], 0.0)
    out_ref[...] = o


def kernel(xyz, feature, sample_idx, W0, b0, W1, b1, W2, b2, V0, c0):
    si = sample_idx.astype(jnp.int32)
    new_xyz = jnp.take_along_axis(xyz, si[:, :, None], axis=1)
    w3 = W0[:, :3].T
    w64 = W0[:, 3:].T

    wspec = lambda shape: pl.BlockSpec(shape, lambda i: (0,) * len(shape))

    g = pl.pallas_call(
        _gpre_body,
        grid=(_BN // _GB,),
        in_specs=[
            pl.BlockSpec((_GB, 3), lambda i: (i, 0)),
            pl.BlockSpec((_GB, _C), lambda i: (i, 0)),
            wspec((3, 64)), wspec((_C, 64)), wspec((1, 64)),
        ],
        out_specs=pl.BlockSpec((_GB, 128), lambda i: (i, 0)),
        out_shape=jax.ShapeDtypeStruct((_BN, 128), jnp.float32),
    )(xyz.reshape(_BN, 3), feature.reshape(_BN, _C), w3, w64, b0[None, :])

    du, qproj = pl.pallas_call(
        _dist_sel_body,
        grid=(_R // _SB,),
        in_specs=[
            pl.BlockSpec((_SB, 3), lambda i: (i, 0)),
            pl.BlockSpec((1, _N, 3), lambda i: (i // (_S // _SB), 0, 0)),
            wspec((3, 64)),
        ],
        out_specs=[
            pl.BlockSpec((_SB, _RW), lambda i: (i, 0)),
            pl.BlockSpec((_SB, 64), lambda i: (i, 0)),
        ],
        out_shape=[
            jax.ShapeDtypeStruct((_R, _RW), jnp.int32),
            jax.ShapeDtypeStruct((_R, 64), jnp.float32),
        ],
    )(new_xyz.reshape(_R, 3), xyz, w3)

    select = functools.partial(
        pl.kernel,
        out_type=jax.ShapeDtypeStruct((_R * _K, 128), jnp.float32),
        mesh=plsc.VectorSubcoreMesh(core_axis_name="c", subcore_axis_name="s"),
        compiler_params=pltpu.CompilerParams(needs_layout_passes=False),
        scratch_types=[
            pltpu.VMEM((2 * _RW,), jnp.int32),
            pltpu.VMEM((16 * _DEPTH + 16,), jnp.int32),
            pltpu.VMEM((16 * _DEPTH + 16,), jnp.int32),
            pltpu.VMEM((2 * _OW,), jnp.int32),
            pltpu.VMEM((2 * _K, 128), jnp.float32),
            pltpu.SemaphoreType.DMA,
            pltpu.SemaphoreType.DMA,
            pltpu.SemaphoreType.DMA,
            pltpu.SemaphoreType.DMA,
            pltpu.SemaphoreType.DMA,
            pltpu.SemaphoreType.DMA,
        ],
    )(_select_body)
    gg = select(du, g)

    new_feature = pl.pallas_call(
        _mlp_body,
        grid=(_R // _SE,),
        in_specs=[
            pl.BlockSpec((_SE * _K, 128), lambda i: (i, 0)),
            pl.BlockSpec((_SE, 64), lambda i: (i, 0)),
            wspec((64, 64)), wspec((1, 64)),
            wspec((64, 128)), wspec((1, 128)),
            wspec((128, 128)), wspec((1, 128)),
        ],
        out_specs=pl.BlockSpec((_SE, 128), lambda i: (i, 0)),
        out_shape=jax.ShapeDtypeStruct((_R, 128), jnp.float32),
    )(gg, qproj, W1.T, b1[None, :], W2.T, b2[None, :], V0.T, c0[None, :])

    return (new_xyz, new_feature.reshape(_B, _S, 128), sample_idx)
